# Initial kernel scaffold; baseline (speedup 1.0000x reference)
import jax
import jax.numpy as jnp
from jax.experimental import pallas as pl


def kernel(cart, atom_species, params, centerlist, neighlist, local_species, neigh_species, center_neighlist):
    raise NotImplementedError("write your pallas kernel here")


# speedup vs baseline: 1.0024x; 1.0024x over previous
import jax, jax.numpy as jnp, numpy as np
from jax.experimental import pallas as pl

NW = 8; NANG = 9; CUT = 4.0
INDEX_L = np.array([0, 1, 1, 1, 2, 2, 2, 2, 2])
c0 = 0.28209479177387814; c1 = 0.4886025119029199
c2 = 1.0925484305920792; c2b = 0.31539156525252005; c2c = 0.5462742152960396


def _mlp_fwd(x, Ws, bs):
    saves = []
    h = x
    for W, b in zip(Ws[:-1], bs[:-1]):
        z = h @ W + b
        m = jnp.mean(z, axis=-1, keepdims=True)
        v = jnp.var(z, axis=-1, keepdims=True)
        s = jnp.sqrt(v + 1e-5)
        zh = (z - m) / s
        saves.append((W, s, zh))
        h = zh * jax.nn.sigmoid(zh)
    out = h @ Ws[-1] + bs[-1]
    saves.append((Ws[-1],))
    return out, saves


def _mlp_bwd(dout, saves):
    dh = dout @ saves[-1][0].T
    for (W, s, zh) in reversed(saves[:-1]):
        sig = jax.nn.sigmoid(zh)
        dzh = dh * sig * (1 + zh * (1 - sig))
        dz = (dzh - jnp.mean(dzh, axis=-1, keepdims=True)
              - zh * jnp.mean(dzh * zh, axis=-1, keepdims=True)) / s
        dh = dz @ W.T
    return dh


def _copy_kernel(x_ref, o_ref):
    o_ref[...] = x_ref[...]


def _pl_copy(x):
    return pl.pallas_call(
        _copy_kernel,
        out_shape=jax.ShapeDtypeStruct(x.shape, x.dtype),
    )(x)


def kernel(cart, atom_species, params, centerlist, neighlist, local_species, neigh_species, center_neighlist):
    N = local_species.shape[0]
    lc = _mlp_fwd(atom_species, params['ec_W'], params['ec_b'])[0][0]
    ne = _mlp_fwd(atom_species, params['en_W'], params['en_b'])[0][0]
    w = ne[:NW]; beta = ne[NW:2 * NW]; mu = ne[2 * NW:]
    cl = centerlist; nl = neighlist; g = center_neighlist
    v = cart[cl] - cart[nl]
    d = jnp.linalg.norm(v, axis=1)
    t = 0.5 * jnp.cos(d * (np.pi / CUT)) + 0.5
    f = t * t
    dm = d[:, None] - mu[None, :]
    rho = jnp.exp(-jnp.square(beta[None, :] * dm))
    x, y, z = v[:, 0], v[:, 1], v[:, 2]
    S = jnp.stack([c0 * jnp.ones_like(x), c1 * y, c1 * z, c1 * x,
                   c2 * x * y, c2 * y * z, c2b * (2 * z * z - x * x - y * y),
                   c2 * x * z, c2c * (x * x - y * y)], axis=1)
    rw = rho * w[None, :]
    O = f[:, None, None] * S[:, :, None] * rw[:, None, :]
    cc = params['contracted_coeff'][:, INDEX_L]
    CO = [None] * 4; A = [None] * 4; D = [None] * 4; Q = [None] * 4; sv = [None] * 4
    CO[0] = jnp.zeros((N, NANG, NW), cart.dtype).at[cl].add(O)
    A[0] = jnp.einsum('ikj,kjm->ikm', CO[0], cc[0])
    D[0] = jnp.einsum('ikm,ikm->im', A[0], A[0]) * lc[None, :]
    for it in range(1, 4):
        Q[it], sv[it] = _mlp_fwd(D[it - 1], params['it_W'][it - 1], params['it_b'][it - 1])
        WO = Q[it][g][:, None, :] * O + f[:, None, None] * CO[it - 1][g]
        CO[it] = CO[it - 1].at[cl].add(WO)
        A[it] = jnp.einsum('ikj,kjm->ikm', CO[it], cc[it])
        D[it] = D[it - 1] + jnp.einsum('ikm,ikm->im', A[it], A[it]) * lc[None, :]
    out, out_sv = _mlp_fwd(D[3], params['out_W'], params['out_b'])
    out = _pl_copy(out)
    energy = jnp.sum(out)

    gD = _mlp_bwd(jnp.ones_like(out), out_sv)
    P = jnp.zeros_like(CO[0])
    dfE = jnp.zeros_like(f)
    dO = jnp.zeros_like(O)
    for it in range(3, 0, -1):
        dA = 2.0 * A[it] * (lc[None, None, :] * gD[:, None, :])
        P = P + jnp.einsum('ikm,kjm->ikj', dA, cc[it])
        T = P[cl]
        dq = jnp.zeros((N, NW), cart.dtype).at[g].add(jnp.einsum('ekj,ekj->ej', T, O))
        dO = dO + Q[it][g][:, None, :] * T
        dfE = dfE + jnp.einsum('ekj,ekj->e', T, CO[it - 1][g])
        P = P + jnp.zeros_like(P).at[g].add(f[:, None, None] * T)
        gD = gD + _mlp_bwd(dq, sv[it])
    dA = 2.0 * A[0] * (lc[None, None, :] * gD[:, None, :])
    P = P + jnp.einsum('ikm,kjm->ikj', dA, cc[0])
    dO = dO + P[cl]

    dfE = dfE + jnp.einsum('ekj,ej,ek->e', dO, rw, S)
    drho = f[:, None] * w[None, :] * jnp.einsum('ekj,ek->ej', dO, S)
    dS = f[:, None] * jnp.einsum('ekj,ej->ek', dO, rw)
    dd = dfE * (2.0 * t) * (-0.5 * np.pi / CUT) * jnp.sin(d * (np.pi / CUT))
    dd = dd + jnp.sum(drho * rho * (-2.0 * beta[None, :] ** 2 * dm), axis=1)
    dx = c1 * dS[:, 3] + c2 * y * dS[:, 4] - 2 * c2b * x * dS[:, 6] + c2 * z * dS[:, 7] + 2 * c2c * x * dS[:, 8]
    dy = c1 * dS[:, 1] + c2 * x * dS[:, 4] + c2 * z * dS[:, 5] - 2 * c2b * y * dS[:, 6] - 2 * c2c * y * dS[:, 8]
    dz = c1 * dS[:, 2] + c2 * y * dS[:, 5] + 4 * c2b * z * dS[:, 6] + c2 * x * dS[:, 7]
    dv = jnp.stack([dx, dy, dz], axis=1) + (dd / d)[:, None] * v
    dcart = jnp.zeros_like(cart).at[cl].add(dv).at[nl].add(-dv)
    return energy, -dcart.reshape(-1), out


# speedup vs baseline: 36.3648x; 36.2777x over previous
import functools
import jax, jax.numpy as jnp, numpy as np
from jax import lax
from jax.experimental import pallas as pl
from jax.experimental.pallas import tpu as pltpu, tpu_sc as plsc

NW = 8; NANG = 9; CUT = 4.0
INDEX_L = np.array([0, 1, 1, 1, 2, 2, 2, 2, 2])
c0 = 0.28209479177387814; c1 = 0.4886025119029199
c2 = 1.0925484305920792; c2b = 0.31539156525252005; c2c = 0.5462742152960396

NC, NS, L = 2, 16, 16
NWK = NC * NS
CH = 16
GW = 24
R = 80
QW = 16

_SCP = pltpu.CompilerParams(needs_layout_passes=False, use_tc_tiling_on_sc=False)
def _mesh():
    return plsc.VectorSubcoreMesh(core_axis_name="c", subcore_axis_name="s",
                                  num_cores=NC, num_subcores=NS)


def _rsqrt(r2):
    y = plsc.bitcast(jnp.int32(0x5F3759DF) - (plsc.bitcast(r2, jnp.int32) >> 1), jnp.float32)
    for _ in range(3):
        y = y * (1.5 - 0.5 * r2 * y * y)
    return y


def _sincos(z):
    k = (z * (2.0 / np.pi) + 0.5).astype(jnp.int32)
    kf = k.astype(jnp.float32)
    r = z - kf * 1.5707855224609375
    r = r - kf * 1.0804334124e-05
    r = r - kf * 2.5579538487e-12
    r2 = r * r
    s = r * (1.0 + r2 * (-0.16666654611 + r2 * (0.0083321608736 + r2 * -0.00019515295891)))
    c = 1.0 + r2 * (-0.5 + r2 * (0.0416666418 + r2 * (-0.001388731625 + r2 * 2.443315711e-05)))
    q = k & 3
    sin = jnp.where(q == 0, s, jnp.where(q == 1, c, jnp.where(q == 2, -s, -c)))
    cos = jnp.where(q == 0, c, jnp.where(q == 1, -s, jnp.where(q == 2, -c, s)))
    return sin, cos


def _wid():
    return lax.axis_index("s") * NC + lax.axis_index("c")


def _zero_acc(zeros_hbm, acc_sh, n_rows):
    sid = lax.axis_index("s")
    stride = n_rows // NS
    pltpu.sync_copy(zeros_hbm.at[pl.ds(sid * stride, stride)],
                    acc_sh.at[pl.ds(sid * stride, stride)])


def _readout_acc(acc_sh, out_hbm, n_rows):
    cid = lax.axis_index("c")
    sid = lax.axis_index("s")
    stride = n_rows // NS
    pltpu.sync_copy(acc_sh.at[pl.ds(sid * stride, stride)],
                    out_hbm.at[cid, pl.ds(sid * stride, stride)])


def _make_sc_kernels(N, E):
    NCHUNK = E // (NWK * CH)
    EPW = E // NWK

    def _lanevecs():
        lanes = lax.iota(jnp.int32, L)
        return (lanes, lanes & 7, lanes >> 3,
                jnp.where(lanes < 8, 1.0, 0.0).astype(jnp.float32))

    @functools.partial(
        pl.kernel, mesh=_mesh(), compiler_params=_SCP,
        out_type=[jax.ShapeDtypeStruct((E * GW,), jnp.float32),
                  jax.ShapeDtypeStruct((NC, N, R), jnp.float32)],
        scratch_types=[pltpu.VMEM((N,), jnp.float32),
                       pltpu.VMEM((N,), jnp.float32),
                       pltpu.VMEM((N,), jnp.float32),
                       pltpu.VMEM((EPW,), jnp.int32),
                       pltpu.VMEM((EPW,), jnp.int32),
                       pltpu.VMEM((EPW,), jnp.int32),
                       pltpu.VMEM((48,), jnp.float32),
                       [pltpu.VMEM((CH,), jnp.int32)] * 2,
                       [pltpu.VMEM((CH,), jnp.int32)] * 2,
                       [pltpu.VMEM((CH, QW), jnp.float32)] * 2,
                       [pltpu.VMEM((CH * GW,), jnp.float32)] * 2,
                       [pltpu.VMEM((CH, R), jnp.float32)] * 2,
                       [pltpu.VMEM((CH, R), jnp.float32)] * 2,
                       pltpu.VMEM_SHARED((N, R), jnp.float32),
                       [pltpu.SemaphoreType.DMA] * 2,
                       [pltpu.SemaphoreType.DMA] * 2,
                       [pltpu.SemaphoreType.DMA] * 2,
                       [pltpu.SemaphoreType.DMA] * 2],
    )
    def edge_fwd(cart_hbm, cl_hbm, nl_hbm, g_hbm, q_hbm, co_hbm, aux_hbm, zeros_hbm,
                 geo_out, part_out,
                 cx_v, cy_v, cz_v, cl1_v, nl1_v, g1_v, aux_v,
                 clidx, gidx, qrows, geo_st, corows, wo,
                 acc_sh, cosem, qsem, scsem, geosem):
        wid = _wid()
        base = wid * NCHUNK
        lanes, j8, khalf, mask4 = _lanevecs()
        pltpu.sync_copy(cart_hbm.at[0], cx_v)
        pltpu.sync_copy(cart_hbm.at[1], cy_v)
        pltpu.sync_copy(cart_hbm.at[2], cz_v)
        pltpu.sync_copy(cl_hbm.at[pl.ds(wid * EPW, EPW)], cl1_v)
        pltpu.sync_copy(nl_hbm.at[pl.ds(wid * EPW, EPW)], nl1_v)
        pltpu.sync_copy(g_hbm.at[pl.ds(wid * EPW, EPW)], g1_v)
        pltpu.sync_copy(aux_hbm, aux_v)
        _zero_acc(zeros_hbm, acc_sh, N)
        plsc.subcore_barrier()

        for par in range(2):
            gidx[par][pl.ds(0, CH)] = g1_v[pl.ds(par * CH, CH)]
            pltpu.async_copy(co_hbm.at[gidx[par]], corows[par], cosem[par])
            pltpu.async_copy(q_hbm.at[gidx[par]], qrows[par], qsem[par])

        def half(ch, par):
            pltpu.make_async_copy(co_hbm.at[gidx[par]], corows[par], cosem[par]).wait()
            pltpu.make_async_copy(q_hbm.at[gidx[par]], qrows[par], qsem[par]).wait()

            @pl.when(ch >= 2)
            def _():
                pltpu.make_async_copy(wo[par], acc_sh.at[clidx[par]], scsem[par]).wait()
                pltpu.make_async_copy(
                    geo_st[par],
                    geo_out.at[pl.ds((base + ch) * CH * GW, CH * GW)],
                    geosem[par]).wait()

            clv = cl1_v[pl.ds(ch * CH, CH)]
            nlv = nl1_v[pl.ds(ch * CH, CH)]
            vx = plsc.load_gather(cx_v, [clv]) - plsc.load_gather(cx_v, [nlv])
            vy = plsc.load_gather(cy_v, [clv]) - plsc.load_gather(cy_v, [nlv])
            vz = plsc.load_gather(cz_v, [clv]) - plsc.load_gather(cz_v, [nlv])
            r2 = jnp.maximum(vx * vx + vy * vy + vz * vz, 1e-30)
            invd = _rsqrt(r2)
            d = r2 * invd
            sh, chh = _sincos(d * (np.pi / (2.0 * CUT)))
            t = chh * chh
            f = t * t
            sd = 2.0 * sh * chh
            gst = geo_st[par]
            plsc.store_scatter(gst, [lanes * GW + 21], f)
            plsc.store_scatter(gst, [lanes * GW + 1], t)
            plsc.store_scatter(gst, [lanes * GW + 2], d)
            plsc.store_scatter(gst, [lanes * GW + 3], invd)
            plsc.store_scatter(gst, [lanes * GW + 0], sd)
            for j in range(NW):
                bj = plsc.load_gather(aux_v, [jnp.full((L,), 1 + j, jnp.int32)])
                mj = plsc.load_gather(aux_v, [jnp.full((L,), 9 + j, jnp.int32)])
                rj = bj * (d - mj)
                rho = jnp.exp(-(rj * rj))
                plsc.store_scatter(gst, [lanes * GW + 4 + j], rho)
            svals = [jnp.full((L,), c0, jnp.float32), c1 * vy, c1 * vz, c1 * vx,
                     c2 * vx * vy, c2 * vy * vz,
                     c2b * (2.0 * vz * vz - vx * vx - vy * vy),
                     c2 * vx * vz, c2c * (vx * vx - vy * vy)]
            for k in range(NANG):
                plsc.store_scatter(gst, [lanes * GW + 12 + k], svals[k])
            pltpu.async_copy(
                gst, geo_out.at[pl.ds((base + ch) * CH * GW, CH * GW)], geosem[par])
            wpat = plsc.load_gather(aux_v, [17 + j8])
            for i in range(CH):
                qv = qrows[par][i, pl.ds(0, QW)]
                fvec = plsc.load_gather(gst, [jnp.full((L,), i * GW + 21, jnp.int32)])
                rhov = plsc.load_gather(gst, [i * GW + 4 + j8])
                frwq = fvec * rhov * wpat * qv
                for sl in range(5):
                    sv = plsc.load_gather(gst, [i * GW + 12 + sl * 2 + khalf])
                    co = corows[par][i, pl.ds(sl * L, L)]
                    w_o = sv * frwq
                    if sl == 4:
                        w_o = w_o * mask4
                    wo[par][i, pl.ds(sl * L, L)] = w_o + fvec * co
            clidx[par][pl.ds(0, CH)] = cl1_v[pl.ds(ch * CH, CH)]
            pltpu.async_copy(wo[par], acc_sh.at[clidx[par]], scsem[par], add=True)

            @pl.when(ch + 2 < NCHUNK)
            def _():
                gidx[par][pl.ds(0, CH)] = g1_v[pl.ds((ch + 2) * CH, CH)]
                pltpu.async_copy(co_hbm.at[gidx[par]], corows[par], cosem[par])
                pltpu.async_copy(q_hbm.at[gidx[par]], qrows[par], qsem[par])

        def body(m, carry):
            half(2 * m, 0)
            ch1 = 2 * m + 1

            @pl.when(ch1 < NCHUNK)
            def _():
                half(ch1, 1)
            return carry

        lax.fori_loop(0, (NCHUNK + 1) // 2, body, 0)
        for par in range(2):
            pltpu.make_async_copy(wo[par], acc_sh.at[clidx[par]], scsem[par]).wait()
            pltpu.make_async_copy(
                geo_st[par], geo_out.at[pl.ds(base * CH * GW, CH * GW)],
                geosem[par]).wait()
        plsc.subcore_barrier()
        _readout_acc(acc_sh, part_out, N)

    def _make_bwd():
        @functools.partial(
            pl.kernel, mesh=_mesh(), compiler_params=_SCP,
            out_type=[jax.ShapeDtypeStruct((NC, N, R), jnp.float32),
                      jax.ShapeDtypeStruct((NC, N, QW), jnp.float32),
                      jax.ShapeDtypeStruct((E * GW,), jnp.float32)],
            scratch_types=[pltpu.VMEM((EPW,), jnp.int32),
                           pltpu.VMEM((EPW,), jnp.int32),
                           pltpu.VMEM((48,), jnp.float32),
                           [pltpu.VMEM((CH,), jnp.int32)] * 2,
                           [pltpu.VMEM((CH,), jnp.int32)] * 2,
                           [pltpu.VMEM((CH,), jnp.int32)] * 2,
                           [pltpu.VMEM((CH, QW), jnp.float32)] * 2,
                           [pltpu.VMEM((CH * GW,), jnp.float32)] * 2,
                           [pltpu.VMEM((CH * GW,), jnp.float32)] * 2,
                           [pltpu.VMEM((CH * GW,), jnp.float32)] * 2,
                           [pltpu.VMEM((CH, R), jnp.float32)] * 2,
                           [pltpu.VMEM((CH, R), jnp.float32)] * 2,
                           [pltpu.VMEM((CH, R), jnp.float32)] * 2,
                           [pltpu.VMEM((CH, QW), jnp.float32)] * 2,
                           pltpu.VMEM((CH * R,), jnp.float32),
                           pltpu.VMEM((CH * R,), jnp.float32),
                           pltpu.VMEM((CH * QW,), jnp.float32),
                           pltpu.VMEM((CH * QW,), jnp.float32),
                           pltpu.VMEM_SHARED((N, R), jnp.float32),
                           pltpu.VMEM_SHARED((N, QW), jnp.float32),
                           [pltpu.SemaphoreType.DMA] * 2,
                           [pltpu.SemaphoreType.DMA] * 2,
                           [pltpu.SemaphoreType.DMA] * 2,
                           [pltpu.SemaphoreType.DMA] * 2,
                           [pltpu.SemaphoreType.DMA] * 2,
                           [pltpu.SemaphoreType.DMA] * 2,
                           [pltpu.SemaphoreType.DMA] * 2,
                           [pltpu.SemaphoreType.DMA] * 2],
        )
        def bwdpass(geo_hbm, cl_hbm, g_hbm, q_hbm, co_hbm, p_hbm,
                    accin_hbm, aux_hbm, zeros_hbm, zerosq_hbm,
                    ft_out, dq_out, acc_out,
                    cl1_v, g1_v, aux_v,
                    clidx, ggidx, sgidx, qrows, geo_st, accin_st, accout_st,
                    trows, corows, ftrows, dqrows,
                    tcm, cocm, qcm, dqcm,
                    pacc_sh, dqacc_sh,
                    clsem, cosem, qsem, gsem, asem, aosem, ftsem, dqsem):
            wid = _wid()
            base = wid * NCHUNK
            lanes, j8, khalf, mask4 = _lanevecs()
            pltpu.sync_copy(cl_hbm.at[pl.ds(wid * EPW, EPW)], cl1_v)
            pltpu.sync_copy(g_hbm.at[pl.ds(wid * EPW, EPW)], g1_v)
            pltpu.sync_copy(aux_hbm, aux_v)
            _zero_acc(zeros_hbm, pacc_sh, N)
            sid = lax.axis_index("s")
            strq = N // NS
            pltpu.sync_copy(zerosq_hbm.at[pl.ds(sid * strq, strq)],
                            dqacc_sh.at[pl.ds(sid * strq, strq)])
            plsc.subcore_barrier()

            def issue_inputs(ch, par):
                clidx[par][pl.ds(0, CH)] = cl1_v[pl.ds(ch * CH, CH)]
                ggidx[par][pl.ds(0, CH)] = g1_v[pl.ds(ch * CH, CH)]
                pltpu.async_copy(p_hbm.at[clidx[par]], trows[par], clsem[par])
                pltpu.async_copy(co_hbm.at[ggidx[par]], corows[par], cosem[par])
                pltpu.async_copy(q_hbm.at[ggidx[par]], qrows[par], qsem[par])
                pltpu.async_copy(
                    geo_hbm.at[pl.ds((base + ch) * CH * GW, CH * GW)],
                    geo_st[par], gsem[par])
                pltpu.async_copy(
                    accin_hbm.at[pl.ds((base + ch) * CH * GW, CH * GW)],
                    accin_st[par], asem[par])

            for par in range(2):
                issue_inputs(par, par)

            def half(ch, par):
                gst = geo_st[par]
                pltpu.make_async_copy(p_hbm.at[clidx[par]], trows[par], clsem[par]).wait()
                pltpu.make_async_copy(co_hbm.at[ggidx[par]], corows[par], cosem[par]).wait()
                pltpu.make_async_copy(q_hbm.at[ggidx[par]], qrows[par], qsem[par]).wait()
                pltpu.make_async_copy(
                    geo_hbm.at[pl.ds(base * CH * GW, CH * GW)], gst, gsem[par]).wait()
                pltpu.make_async_copy(
                    accin_hbm.at[pl.ds(base * CH * GW, CH * GW)],
                    accin_st[par], asem[par]).wait()

                @pl.when(ch >= 2)
                def _():
                    pltpu.make_async_copy(
                        accout_st[par],
                        acc_out.at[pl.ds(base * CH * GW, CH * GW)], aosem[par]).wait()
                    pltpu.make_async_copy(
                        ftrows[par], pacc_sh.at[sgidx[par]], ftsem[par]).wait()
                    pltpu.make_async_copy(
                        dqrows[par], dqacc_sh.at[sgidx[par]], dqsem[par]).wait()

                for i in range(CH):
                    for sl in range(5):
                        tv = trows[par][i, pl.ds(sl * L, L)]
                        plsc.store_scatter(tcm, [(sl * L + lanes) * CH + i], tv)
                        cv = corows[par][i, pl.ds(sl * L, L)]
                        plsc.store_scatter(cocm, [(sl * L + lanes) * CH + i], cv)
                    qrv = qrows[par][i, pl.ds(0, QW)]
                    plsc.store_scatter(qcm, [lanes * CH + i], qrv)
                for i in range(CH):
                    fsp = plsc.load_gather(gst, [jnp.full((L,), i * GW + 21, jnp.int32)])
                    for sl in range(5):
                        ftrows[par][i, pl.ds(sl * L, L)] = fsp * trows[par][i, pl.ds(sl * L, L)]
                fcol = plsc.load_gather(gst, [lanes * GW + 21])
                scols = [plsc.load_gather(gst, [lanes * GW + 12 + k]) for k in range(NANG)]
                rwcols = []
                qcols = []
                frwcols = []
                for j in range(NW):
                    rho = plsc.load_gather(gst, [lanes * GW + 4 + j])
                    wsp = plsc.load_gather(aux_v, [jnp.full((L,), 17 + j, jnp.int32)])
                    rw = rho * wsp
                    rwcols.append(rw)
                    frwcols.append(fcol * rw)
                    qcols.append(qcm[pl.ds(j * CH, CH)])
                dfE = jnp.zeros((L,), jnp.float32)
                uu = [jnp.zeros((L,), jnp.float32) for _ in range(NW)]
                ww = [jnp.zeros((L,), jnp.float32) for _ in range(NANG)]
                dq = [jnp.zeros((L,), jnp.float32) for _ in range(NW)]
                for k in range(NANG):
                    for j in range(NW):
                        c = k * NW + j
                        Tc = tcm[pl.ds(c * CH, CH)]
                        COc = cocm[pl.ds(c * CH, CH)]
                        dfE = dfE + Tc * COc
                        ts = Tc * scols[k]
                        uu[j] = uu[j] + qcols[j] * ts
                        dq[j] = dq[j] + ts * frwcols[j]
                        ww[k] = ww[k] + (qcols[j] * Tc) * rwcols[j]
                a0 = plsc.load_gather(accin_st[par], [lanes * GW + 0])
                plsc.store_scatter(accout_st[par], [lanes * GW + 0], a0 + dfE)
                for j in range(NW):
                    aj = plsc.load_gather(accin_st[par], [lanes * GW + 1 + j])
                    plsc.store_scatter(accout_st[par], [lanes * GW + 1 + j], aj + uu[j])
                for k in range(NANG):
                    ak = plsc.load_gather(accin_st[par], [lanes * GW + 9 + k])
                    plsc.store_scatter(accout_st[par], [lanes * GW + 9 + k], ak + ww[k])
                pltpu.async_copy(
                    accout_st[par],
                    acc_out.at[pl.ds((base + ch) * CH * GW, CH * GW)], aosem[par])
                for j in range(NW):
                    plsc.store_scatter(dqcm, [lanes * QW + j], dq[j])
                for i in range(CH):
                    dqrows[par][i, pl.ds(0, QW)] = dqcm[pl.ds(i * QW, QW)]
                sgidx[par][pl.ds(0, CH)] = g1_v[pl.ds(ch * CH, CH)]
                pltpu.async_copy(ftrows[par], pacc_sh.at[sgidx[par]], ftsem[par], add=True)
                pltpu.async_copy(dqrows[par], dqacc_sh.at[sgidx[par]], dqsem[par], add=True)

                @pl.when(ch + 2 < NCHUNK)
                def _():
                    issue_inputs(ch + 2, par)

            def body(m, carry):
                half(2 * m, 0)
                ch1 = 2 * m + 1

                @pl.when(ch1 < NCHUNK)
                def _():
                    half(ch1, 1)
                return carry

            lax.fori_loop(0, (NCHUNK + 1) // 2, body, 0)
            for par in range(2):
                pltpu.make_async_copy(
                    accout_st[par],
                    acc_out.at[pl.ds(base * CH * GW, CH * GW)], aosem[par]).wait()
                pltpu.make_async_copy(
                    ftrows[par], pacc_sh.at[sgidx[par]], ftsem[par]).wait()
                pltpu.make_async_copy(
                    dqrows[par], dqacc_sh.at[sgidx[par]], dqsem[par]).wait()
            plsc.subcore_barrier()
            _readout_acc(pacc_sh, ft_out, N)
            cid = lax.axis_index("c")
            pltpu.sync_copy(dqacc_sh.at[pl.ds(sid * strq, strq)],
                            dq_out.at[cid, pl.ds(sid * strq, strq)])
        return bwdpass

    bwdpass = _make_bwd()

    @functools.partial(
        pl.kernel, mesh=_mesh(), compiler_params=_SCP,
        out_type=jax.ShapeDtypeStruct((NC, N, QW), jnp.float32),
        scratch_types=[pltpu.VMEM((N,), jnp.float32),
                       pltpu.VMEM((N,), jnp.float32),
                       pltpu.VMEM((N,), jnp.float32),
                       pltpu.VMEM((CH,), jnp.int32),
                       pltpu.VMEM((CH,), jnp.int32),
                       pltpu.VMEM((EPW,), jnp.int32),
                       pltpu.VMEM((EPW,), jnp.int32),
                       pltpu.VMEM((48,), jnp.float32),
                       pltpu.VMEM((CH * GW,), jnp.float32),
                       pltpu.VMEM((CH * GW,), jnp.float32),
                       pltpu.VMEM((CH, R), jnp.float32),
                       pltpu.VMEM((CH * R,), jnp.float32),
                       pltpu.VMEM((CH * QW,), jnp.float32),
                       pltpu.VMEM((CH, QW), jnp.float32),
                       pltpu.VMEM((CH, QW), jnp.float32),
                       pltpu.VMEM_SHARED((N, QW), jnp.float32),
                       pltpu.SemaphoreType.DMA,
                       pltpu.SemaphoreType.DMA,
                       pltpu.SemaphoreType.DMA],
    )
    def finalpass(cart_hbm, geo_hbm, acc_hbm, p0_hbm, cl_hbm,
                  nl_hbm, aux_hbm, zerosq_hbm,
                  dcart_out,
                  cx_v, cy_v, cz_v, clidx_v, nlidx_v, cl1_v, nl1_v, aux_v,
                  geo_st, acc_st, prows_v, pcm, dvcm, dvrows_v, ndvrows_v,
                  dacc_sh, sem, sem2, sem3):
        wid = _wid()
        base = wid * NCHUNK
        lanes, j8, khalf, mask4 = _lanevecs()
        pltpu.sync_copy(cart_hbm.at[0], cx_v)
        pltpu.sync_copy(cart_hbm.at[1], cy_v)
        pltpu.sync_copy(cart_hbm.at[2], cz_v)
        pltpu.sync_copy(cl_hbm.at[pl.ds(wid * EPW, EPW)], cl1_v)
        pltpu.sync_copy(nl_hbm.at[pl.ds(wid * EPW, EPW)], nl1_v)
        pltpu.sync_copy(aux_hbm, aux_v)
        sid = lax.axis_index("s")
        strq = N // NS
        pltpu.sync_copy(zerosq_hbm.at[pl.ds(sid * strq, strq)],
                        dacc_sh.at[pl.ds(sid * strq, strq)])
        plsc.subcore_barrier()

        def body(ch, carry):
            clidx_v[pl.ds(0, CH)] = cl1_v[pl.ds(ch * CH, CH)]
            nlidx_v[pl.ds(0, CH)] = nl1_v[pl.ds(ch * CH, CH)]
            cp1 = pltpu.async_copy(p0_hbm.at[clidx_v], prows_v, sem)
            cp2 = pltpu.async_copy(
                geo_hbm.at[pl.ds((base + ch) * CH * GW, CH * GW)], geo_st, sem2)
            cp3 = pltpu.async_copy(
                acc_hbm.at[pl.ds((base + ch) * CH * GW, CH * GW)], acc_st, sem3)
            cp1.wait()
            cp2.wait()
            cp3.wait()
            for i in range(CH):
                for sl in range(5):
                    pv = prows_v[i, pl.ds(sl * L, L)]
                    plsc.store_scatter(pcm, [(sl * L + lanes) * CH + i], pv)
            clv = cl1_v[pl.ds(ch * CH, CH)]
            nlv = nl1_v[pl.ds(ch * CH, CH)]
            vx = plsc.load_gather(cx_v, [clv]) - plsc.load_gather(cx_v, [nlv])
            vy = plsc.load_gather(cy_v, [clv]) - plsc.load_gather(cy_v, [nlv])
            vz = plsc.load_gather(cz_v, [clv]) - plsc.load_gather(cz_v, [nlv])
            fcol = plsc.load_gather(geo_st, [lanes * GW + 21])
            tcol = plsc.load_gather(geo_st, [lanes * GW + 1])
            dcol = plsc.load_gather(geo_st, [lanes * GW + 2])
            invd = plsc.load_gather(geo_st, [lanes * GW + 3])
            sdcol = plsc.load_gather(geo_st, [lanes * GW + 0])
            scols = [plsc.load_gather(geo_st, [lanes * GW + 12 + k]) for k in range(NANG)]
            rhocols = [plsc.load_gather(geo_st, [lanes * GW + 4 + j]) for j in range(NW)]
            uu = [plsc.load_gather(acc_st, [lanes * GW + 1 + j]) for j in range(NW)]
            ww = [plsc.load_gather(acc_st, [lanes * GW + 9 + k]) for k in range(NANG)]
            dfE = plsc.load_gather(acc_st, [lanes * GW + 0])
            for k in range(NANG):
                for j in range(NW):
                    c = k * NW + j
                    Pc = pcm[pl.ds(c * CH, CH)]
                    uu[j] = uu[j] + Pc * scols[k]
                    wsp = plsc.load_gather(aux_v, [jnp.full((L,), 17 + j, jnp.int32)])
                    ww[k] = ww[k] + Pc * (rhocols[j] * wsp)
            dd = jnp.zeros((L,), jnp.float32)
            for j in range(NW):
                wsp = plsc.load_gather(aux_v, [jnp.full((L,), 17 + j, jnp.int32)])
                rw = rhocols[j] * wsp
                dfE = dfE + uu[j] * rw
                wb2 = plsc.load_gather(aux_v, [jnp.full((L,), 25 + j, jnp.int32)])
                mj = plsc.load_gather(aux_v, [jnp.full((L,), 9 + j, jnp.int32)])
                dd = dd + fcol * uu[j] * wb2 * rhocols[j] * (dcol - mj)
            dd = dd + dfE * (-np.pi / CUT) * tcol * sdcol
            dS = [fcol * ww[k] for k in range(NANG)]
            dx = c1 * dS[3] + c2 * vy * dS[4] - 2 * c2b * vx * dS[6] + c2 * vz * dS[7] + 2 * c2c * vx * dS[8]
            dy = c1 * dS[1] + c2 * vx * dS[4] + c2 * vz * dS[5] - 2 * c2b * vy * dS[6] - 2 * c2c * vy * dS[8]
            dz = c1 * dS[2] + c2 * vy * dS[5] + 4 * c2b * vz * dS[6] + c2 * vx * dS[7]
            ddn = dd * invd
            dx = dx + ddn * vx
            dy = dy + ddn * vy
            dz = dz + ddn * vz
            plsc.store_scatter(dvcm, [lanes * QW + 0], dx)
            plsc.store_scatter(dvcm, [lanes * QW + 1], dy)
            plsc.store_scatter(dvcm, [lanes * QW + 2], dz)
            for q in range(3, QW):
                plsc.store_scatter(dvcm, [lanes * QW + q], jnp.zeros((L,), jnp.float32))
            for i in range(CH):
                rv = dvcm[pl.ds(i * QW, QW)]
                dvrows_v[i, pl.ds(0, QW)] = rv
                ndvrows_v[i, pl.ds(0, QW)] = -rv
            pltpu.sync_copy(dvrows_v, dacc_sh.at[clidx_v], add=True)
            pltpu.sync_copy(ndvrows_v, dacc_sh.at[nlidx_v], add=True)
            return carry

        lax.fori_loop(0, NCHUNK, body, 0)
        plsc.subcore_barrier()
        cid = lax.axis_index("c")
        pltpu.sync_copy(dacc_sh.at[pl.ds(sid * strq, strq)],
                        dcart_out.at[cid, pl.ds(sid * strq, strq)])

    return edge_fwd, bwdpass, finalpass


def _mlp_fwd(x, Ws, bs):
    saves = []
    h = x
    for W, b in zip(Ws[:-1], bs[:-1]):
        z = h @ W + b
        m = jnp.mean(z, axis=-1, keepdims=True)
        v = jnp.var(z, axis=-1, keepdims=True)
        s = jnp.sqrt(v + 1e-5)
        zh = (z - m) / s
        saves.append((W, s, zh))
        h = zh * jax.nn.sigmoid(zh)
    out = h @ Ws[-1] + bs[-1]
    saves.append((Ws[-1],))
    return out, saves


def _mlp_bwd(dout, saves):
    dh = dout @ saves[-1][0].T
    for (W, s, zh) in reversed(saves[:-1]):
        sig = jax.nn.sigmoid(zh)
        dzh = dh * sig * (1 + zh * (1 - sig))
        dz = (dzh - jnp.mean(dzh, axis=-1, keepdims=True)
              - zh * jnp.mean(dzh * zh, axis=-1, keepdims=True)) / s
        dh = dz @ W.T
    return dh


def kernel(cart, atom_species, params, centerlist, neighlist, local_species, neigh_species, center_neighlist):
    N = local_species.shape[0]
    E = centerlist.shape[0]
    edge_fwd, bwdpass, finalpass = _make_sc_kernels(N, E)

    lc = _mlp_fwd(atom_species, params['ec_W'], params['ec_b'])[0][0]
    ne = _mlp_fwd(atom_species, params['en_W'], params['en_b'])[0][0]
    w = ne[:NW]; beta = ne[NW:2 * NW]; mu = ne[2 * NW:]
    aux = jnp.zeros((48,), jnp.float32)
    aux = aux.at[1:9].set(beta).at[9:17].set(mu).at[17:25].set(w)
    aux = aux.at[25:33].set(w * (-2.0 * beta * beta))

    i32 = jnp.int32
    cl = centerlist.astype(i32); nl = neighlist.astype(i32); g = center_neighlist.astype(i32)
    cart_t = cart.T
    zeros80 = jnp.zeros((N, R), jnp.float32)
    zeros16 = jnp.zeros((N, QW), jnp.float32)

    ones_q = jnp.ones((N, QW), jnp.float32)
    geo, p0 = edge_fwd(cart_t, cl, nl, g, ones_q, zeros80, aux, zeros80)
    CO = [None] * 4; A = [None] * 4; D = [None] * 4; Q = [None] * 4; sv = [None] * 4
    cc = params['contracted_coeff'][:, INDEX_L]
    CO[0] = (p0[0] + p0[1]).at[:, 72:].set(0.0)

    def contract(co80, t):
        co72 = co80[:, :72].reshape(N, NANG, NW)
        return jnp.einsum('ikj,kjm->ikm', co72, cc[t])

    A[0] = contract(CO[0], 0)
    D[0] = jnp.einsum('ikm,ikm->im', A[0], A[0]) * lc[None, :]
    for t in range(1, 4):
        Q[t], sv[t] = _mlp_fwd(D[t - 1], params['it_W'][t - 1], params['it_b'][t - 1])
        q2 = jnp.tile(Q[t], (1, 2))
        _, pt = edge_fwd(cart_t, cl, nl, g, q2, CO[t - 1], aux, zeros80)
        CO[t] = (CO[t - 1] + pt[0] + pt[1]).at[:, 72:].set(0.0)
        A[t] = contract(CO[t], t)
        D[t] = D[t - 1] + jnp.einsum('ikm,ikm->im', A[t], A[t]) * lc[None, :]
    out, out_sv = _mlp_fwd(D[3], params['out_W'], params['out_b'])
    energy = jnp.sum(out)

    gD = _mlp_bwd(jnp.ones_like(out), out_sv)
    P = jnp.zeros((N, R), jnp.float32)
    ACC = None
    for t in range(3, 0, -1):
        dA = 2.0 * A[t] * (lc[None, None, :] * gD[:, None, :])
        Pc = jnp.einsum('ikm,kjm->ikj', dA, cc[t]).reshape(N, 72)
        P = P.at[:, :72].add(Pc)
        qpad = jnp.tile(Q[t], (1, 2))
        if ACC is None:
            ACC = jnp.zeros((E * GW,), jnp.float32)
        ftp, dqp, ACC = bwdpass(geo, cl, g, qpad, CO[t - 1], P,
                                ACC, aux, zeros80, zeros16)
        dq = (dqp[0] + dqp[1])[:, :NW]
        gD = gD + _mlp_bwd(dq, sv[t])
        P = (P + ftp[0] + ftp[1]).at[:, 72:].set(0.0)
    dA = 2.0 * A[0] * (lc[None, None, :] * gD[:, None, :])
    Pc = jnp.einsum('ikm,kjm->ikj', dA, cc[0]).reshape(N, 72)
    P = P.at[:, :72].add(Pc)

    dcp = finalpass(cart_t, geo, ACC, P, cl, nl, aux, zeros16)
    dcart = (dcp[0] + dcp[1])[:, :3]
    return energy, -dcart.reshape(-1), out
